# baseline (device time: 27864 ns/iter reference)
import jax
import jax.numpy as jnp
from jax import lax
from jax.experimental import pallas as pl
from jax.experimental.pallas import tpu as pltpu

N_DEV = 4
M = 1024
N = 1024
H = M // 2
Q = M // 4
NSUB = 4
NL = 2 * NSUB
CH = N // NL

F32 = jnp.float32
BF16 = jnp.bfloat16


def _gelu(z):
    return 0.5 * z * (1.0 + jnp.tanh(0.7978845608 * (z + 0.044715 * z * z * z)))


def kernel(A, B):
    def body(
        a_ref,
        b_ref,
        out_ref,
        h_send,
        h_recv,
        q_send,
        q_recv,
        send_sems,
        recv_sems,
    ):
        d = lax.axis_index("i")
        p1 = d ^ 1
        p2 = 3 - d

        keep0 = (d ^ (d >> 1)) & 1
        qi0 = keep0 * 2 + (d >> 1)
        keep1 = d >> 1
        qi1 = keep1 * 2 + (d & 1)

        lanes = []
        gc = [(g, c) for c in range(NSUB) for g in (0, 1)]
        for li, (g, c) in enumerate(gc):
            keep = keep0 if g == 0 else keep1
            qi = qi0 if g == 0 else qi1
            qo = keep * 2 + (1 - (qi - keep * 2))
            lanes.append(
                dict(
                    li=li,
                    pa=p1 if g == 0 else p2,
                    pb=p2 if g == 0 else p1,
                    keep_r=keep * H,
                    send_r=(1 - keep) * H,
                    qi_r=qi * Q,
                    off_qi=(qi - keep * 2) * Q,
                    off_qo=(1 - (qi - keep * 2)) * Q,
                    fpo=((1 - (qi - keep * 2)) if g == 0 else (qi - keep * 2)) * Q,
                    col=g * (NSUB * CH) + c * CH,
                )
            )

        barrier_sem = pltpu.get_barrier_semaphore()
        for nbr in [p1, p2]:
            pl.semaphore_signal(
                barrier_sem,
                inc=1,
                device_id=(nbr,),
                device_id_type=pl.DeviceIdType.MESH,
            )
        pl.semaphore_wait(barrier_sem, 2)

        def mm(r, nrows, c):
            a = a_ref[pl.ds(r, nrows), :].astype(BF16)
            b = b_ref[:, pl.ds(c, CH)].astype(BF16)
            return jnp.dot(a, b, preferred_element_type=F32)

        rdma1a = []
        rdma1b = []
        for ln in lanes:
            li = ln["li"]
            h_send[li] = mm(ln["send_r"], H, ln["col"]).astype(BF16)
            fpo2 = Q - ln["fpo"]
            ra = pltpu.make_async_remote_copy(
                src_ref=h_send.at[li, pl.ds(ln["fpo"], Q), :],
                dst_ref=h_recv.at[li, pl.ds(ln["fpo"], Q), :],
                send_sem=send_sems.at[li, 0],
                recv_sem=recv_sems.at[li, 0],
                device_id=(ln["pa"],),
                device_id_type=pl.DeviceIdType.MESH,
            )
            ra.start()
            rb = pltpu.make_async_remote_copy(
                src_ref=h_send.at[li, pl.ds(fpo2, Q), :],
                dst_ref=h_recv.at[li, pl.ds(fpo2, Q), :],
                send_sem=send_sems.at[li, 1],
                recv_sem=recv_sems.at[li, 1],
                device_id=(ln["pa"],),
                device_id_type=pl.DeviceIdType.MESH,
            )
            rb.start()
            rdma1a.append(ra)
            rdma1b.append(rb)
        mm_qo = [mm(ln["keep_r"] + ln["off_qo"], Q, ln["col"]) for ln in lanes]
        mm_qi = [mm(ln["qi_r"], Q, ln["col"]) for ln in lanes]

        rdma2 = []
        for ln, r1a in zip(lanes, rdma1a):
            r1a.wait()
            li = ln["li"]
            q_send[li] = (
                mm_qo[li] + h_recv[li, pl.ds(ln["off_qo"], Q), :].astype(F32)
            ).astype(BF16)
            r = pltpu.make_async_remote_copy(
                src_ref=q_send.at[li],
                dst_ref=q_recv.at[li],
                send_sem=send_sems.at[li, 2],
                recv_sem=recv_sems.at[li, 2],
                device_id=(ln["pb"],),
                device_id_type=pl.DeviceIdType.MESH,
            )
            r.start()
            rdma2.append(r)

        rdma3 = []
        rdma4a = []
        for ln, r1b, r2 in zip(lanes, rdma1b, rdma2):
            li = ln["li"]
            r1b.wait()
            zqp = mm_qi[li] + h_recv[li, pl.ds(ln["off_qi"], Q), :].astype(F32)
            r2.wait()
            gq = _gelu(zqp + q_recv[li].astype(F32))
            qs = (pl.ds(ln["qi_r"], Q), pl.ds(ln["col"], CH))
            out_ref[qs] = gq.astype(BF16)
            r = pltpu.make_async_remote_copy(
                src_ref=out_ref.at[qs],
                dst_ref=out_ref.at[qs],
                send_sem=send_sems.at[li, 3],
                recv_sem=recv_sems.at[li, 3],
                device_id=(ln["pb"],),
                device_id_type=pl.DeviceIdType.MESH,
            )
            r.start()
            rdma3.append(r)
            ra = pltpu.make_async_remote_copy(
                src_ref=out_ref.at[qs],
                dst_ref=out_ref.at[qs],
                send_sem=send_sems.at[li, 4],
                recv_sem=recv_sems.at[li, 4],
                device_id=(ln["pa"],),
                device_id_type=pl.DeviceIdType.MESH,
            )
            ra.start()
            rdma4a.append(ra)

        rdma4b = []
        for ln, r3 in zip(lanes, rdma3):
            r3.wait()
            li = ln["li"]
            os_ = (pl.ds(ln["keep_r"] + ln["off_qo"], Q), pl.ds(ln["col"], CH))
            r = pltpu.make_async_remote_copy(
                src_ref=out_ref.at[os_],
                dst_ref=out_ref.at[os_],
                send_sem=send_sems.at[li, 5],
                recv_sem=recv_sems.at[li, 5],
                device_id=(ln["pa"],),
                device_id_type=pl.DeviceIdType.MESH,
            )
            r.start()
            rdma4b.append(r)
        for ra, rb in zip(rdma4a, rdma4b):
            ra.wait()
            rb.wait()

    return pl.pallas_call(
        body,
        out_shape=jax.ShapeDtypeStruct((M, N), BF16),
        in_specs=[
            pl.BlockSpec(memory_space=pltpu.VMEM),
            pl.BlockSpec(memory_space=pltpu.VMEM),
        ],
        out_specs=pl.BlockSpec(memory_space=pltpu.VMEM),
        scratch_shapes=[
            pltpu.VMEM((NL, H, CH), BF16),
            pltpu.VMEM((NL, H, CH), BF16),
            pltpu.VMEM((NL, Q, CH), BF16),
            pltpu.VMEM((NL, Q, CH), BF16),
            pltpu.SemaphoreType.DMA((NL, 6)),
            pltpu.SemaphoreType.DMA((NL, 6)),
        ],
        compiler_params=pltpu.CompilerParams(collective_id=0),
    )(A, B)


# device time: 27411 ns/iter; 1.0165x vs baseline; 1.0165x over previous
import jax
import jax.numpy as jnp
from jax import lax
from jax.experimental import pallas as pl
from jax.experimental.pallas import tpu as pltpu

N_DEV = 4
M = 1024
N = 1024
H = M // 2
Q = M // 4
NSUB = 4
NL = 2 * NSUB
CH = N // NL

F32 = jnp.float32
BF16 = jnp.bfloat16


def _gelu(z):
    return 0.5 * z * (1.0 + jnp.tanh(0.7978845608 * (z + 0.044715 * z * z * z)))


def kernel(A, B):
    def body(
        a_ref,
        b_ref,
        out_ref,
        h_send,
        h_recv,
        q_send,
        q_recv,
        send_sems,
        recv_sems,
    ):
        d = lax.axis_index("i")
        p1 = d ^ 1
        p2 = 3 - d

        keep0 = (d ^ (d >> 1)) & 1
        qi0 = keep0 * 2 + (d >> 1)
        keep1 = d >> 1
        qi1 = keep1 * 2 + (d & 1)

        lanes = []
        gc = [(g, c) for c in range(NSUB) for g in (0, 1)]
        for li, (g, c) in enumerate(gc):
            keep = keep0 if g == 0 else keep1
            qi = qi0 if g == 0 else qi1
            qo = keep * 2 + (1 - (qi - keep * 2))
            lanes.append(
                dict(
                    li=li,
                    pa=p1 if g == 0 else p2,
                    pb=p2 if g == 0 else p1,
                    keep_r=keep * H,
                    send_r=(1 - keep) * H,
                    qi_r=qi * Q,
                    off_qi=(qi - keep * 2) * Q,
                    off_qo=(1 - (qi - keep * 2)) * Q,
                    fpo=((1 - (qi - keep * 2)) if g == 0 else (qi - keep * 2)) * Q,
                    col=g * (NSUB * CH) + c * CH,
                )
            )

        barrier_sem = pltpu.get_barrier_semaphore()
        for nbr in [p1, p2]:
            pl.semaphore_signal(
                barrier_sem,
                inc=1,
                device_id=(nbr,),
                device_id_type=pl.DeviceIdType.MESH,
            )

        def mm(r, nrows, c):
            a = a_ref[pl.ds(r, nrows), :].astype(BF16)
            b = b_ref[:, pl.ds(c, CH)].astype(BF16)
            return jnp.dot(a, b, preferred_element_type=F32)

        rdma1a = []
        rdma1b = []
        for ln in lanes:
            li = ln["li"]
            h_send[li] = mm(ln["send_r"], H, ln["col"]).astype(BF16)
            if ln["li"] == 0:
                pl.semaphore_wait(barrier_sem, 2)
            fpo2 = Q - ln["fpo"]
            ra = pltpu.make_async_remote_copy(
                src_ref=h_send.at[li, pl.ds(ln["fpo"], Q), :],
                dst_ref=h_recv.at[li, pl.ds(ln["fpo"], Q), :],
                send_sem=send_sems.at[li, 0],
                recv_sem=recv_sems.at[li, 0],
                device_id=(ln["pa"],),
                device_id_type=pl.DeviceIdType.MESH,
            )
            ra.start()
            rb = pltpu.make_async_remote_copy(
                src_ref=h_send.at[li, pl.ds(fpo2, Q), :],
                dst_ref=h_recv.at[li, pl.ds(fpo2, Q), :],
                send_sem=send_sems.at[li, 1],
                recv_sem=recv_sems.at[li, 1],
                device_id=(ln["pa"],),
                device_id_type=pl.DeviceIdType.MESH,
            )
            rb.start()
            rdma1a.append(ra)
            rdma1b.append(rb)
        mm_qo = [mm(ln["keep_r"] + ln["off_qo"], Q, ln["col"]) for ln in lanes]
        mm_qi = [mm(ln["qi_r"], Q, ln["col"]) for ln in lanes]

        rdma2 = []
        for ln, r1a in zip(lanes, rdma1a):
            r1a.wait()
            li = ln["li"]
            q_send[li] = (
                mm_qo[li] + h_recv[li, pl.ds(ln["off_qo"], Q), :].astype(F32)
            ).astype(BF16)
            r = pltpu.make_async_remote_copy(
                src_ref=q_send.at[li],
                dst_ref=q_recv.at[li],
                send_sem=send_sems.at[li, 2],
                recv_sem=recv_sems.at[li, 2],
                device_id=(ln["pb"],),
                device_id_type=pl.DeviceIdType.MESH,
            )
            r.start()
            rdma2.append(r)

        rdma3 = []
        rdma4a = []
        for ln, r1b, r2 in zip(lanes, rdma1b, rdma2):
            li = ln["li"]
            r1b.wait()
            zqp = mm_qi[li] + h_recv[li, pl.ds(ln["off_qi"], Q), :].astype(F32)
            r2.wait()
            gq = _gelu(zqp + q_recv[li].astype(F32))
            qs = (pl.ds(ln["qi_r"], Q), pl.ds(ln["col"], CH))
            out_ref[qs] = gq.astype(BF16)
            r = pltpu.make_async_remote_copy(
                src_ref=out_ref.at[qs],
                dst_ref=out_ref.at[qs],
                send_sem=send_sems.at[li, 3],
                recv_sem=recv_sems.at[li, 3],
                device_id=(ln["pb"],),
                device_id_type=pl.DeviceIdType.MESH,
            )
            r.start()
            rdma3.append(r)
            ra = pltpu.make_async_remote_copy(
                src_ref=out_ref.at[qs],
                dst_ref=out_ref.at[qs],
                send_sem=send_sems.at[li, 4],
                recv_sem=recv_sems.at[li, 4],
                device_id=(ln["pa"],),
                device_id_type=pl.DeviceIdType.MESH,
            )
            ra.start()
            rdma4a.append(ra)

        rdma4b = []
        for ln, r3 in zip(lanes, rdma3):
            r3.wait()
            li = ln["li"]
            os_ = (pl.ds(ln["keep_r"] + ln["off_qo"], Q), pl.ds(ln["col"], CH))
            r = pltpu.make_async_remote_copy(
                src_ref=out_ref.at[os_],
                dst_ref=out_ref.at[os_],
                send_sem=send_sems.at[li, 5],
                recv_sem=recv_sems.at[li, 5],
                device_id=(ln["pa"],),
                device_id_type=pl.DeviceIdType.MESH,
            )
            r.start()
            rdma4b.append(r)
        for ra, rb in zip(rdma4a, rdma4b):
            ra.wait()
            rb.wait()

    return pl.pallas_call(
        body,
        out_shape=jax.ShapeDtypeStruct((M, N), BF16),
        in_specs=[
            pl.BlockSpec(memory_space=pltpu.VMEM),
            pl.BlockSpec(memory_space=pltpu.VMEM),
        ],
        out_specs=pl.BlockSpec(memory_space=pltpu.VMEM),
        scratch_shapes=[
            pltpu.VMEM((NL, H, CH), BF16),
            pltpu.VMEM((NL, H, CH), BF16),
            pltpu.VMEM((NL, Q, CH), BF16),
            pltpu.VMEM((NL, Q, CH), BF16),
            pltpu.SemaphoreType.DMA((NL, 6)),
            pltpu.SemaphoreType.DMA((NL, 6)),
        ],
        compiler_params=pltpu.CompilerParams(collective_id=0),
    )(A, B)
